# baseline (speedup 1.0000x reference)
import functools
import math

import jax
import jax.numpy as jnp
from jax import lax
from jax.experimental import pallas as pl
from jax.experimental.pallas import tpu as pltpu
from jax.experimental.pallas import tpu_sc as plsc

VOCAB = 1000000
EMBED = 64
NUM_SAMPLED = 8192
BATCH = 4096

_NC = 2
_NS = 16
_NW = _NC * _NS
_TRUE_PER_W = BATCH // _NW
_SAMP_PER_W = NUM_SAMPLED // _NW

_LOGV1 = math.log(float(VOCAB) + 1.0)


def _sc_gather(weights, biases, labels_i32, sampled_i32):
    mesh = plsc.VectorSubcoreMesh(core_axis_name="c", subcore_axis_name="s")

    @functools.partial(
        pl.kernel,
        mesh=mesh,
        out_type=[
            jax.ShapeDtypeStruct((BATCH, EMBED), jnp.float32),
            jax.ShapeDtypeStruct((NUM_SAMPLED, EMBED), jnp.float32),
            jax.ShapeDtypeStruct((BATCH,), jnp.float32),
            jax.ShapeDtypeStruct((NUM_SAMPLED,), jnp.float32),
        ],
        scratch_types=[
            pltpu.VMEM((_TRUE_PER_W,), jnp.int32),
            pltpu.VMEM((_SAMP_PER_W,), jnp.int32),
            pltpu.VMEM((_TRUE_PER_W, EMBED), jnp.float32),
            pltpu.VMEM((_SAMP_PER_W, EMBED), jnp.float32),
            pltpu.VMEM((_TRUE_PER_W,), jnp.float32),
            pltpu.VMEM((_SAMP_PER_W,), jnp.float32),
            pltpu.SemaphoreType.DMA,
            pltpu.SemaphoreType.DMA,
            pltpu.SemaphoreType.DMA,
            pltpu.SemaphoreType.DMA,
        ],
        compiler_params=pltpu.CompilerParams(use_tc_tiling_on_sc=False),
    )
    def gather_kernel(w_hbm, b_hbm, lbl_hbm, smp_hbm,
                      tw_out, sw_out, tb_out, sb_out,
                      lbl_v, smp_v, tw_v, sw_v, tb_v, sb_v,
                      sem1, sem2, sem3, sem4):
        wid = lax.axis_index("s") * _NC + lax.axis_index("c")
        bt = wid * _TRUE_PER_W
        bs = wid * _SAMP_PER_W
        pltpu.sync_copy(lbl_hbm.at[pl.ds(bt, _TRUE_PER_W)], lbl_v)
        pltpu.sync_copy(smp_hbm.at[pl.ds(bs, _SAMP_PER_W)], smp_v)
        c1 = pltpu.async_copy(w_hbm.at[lbl_v], tw_v, sem1)
        c2 = pltpu.async_copy(w_hbm.at[smp_v], sw_v, sem2)
        c3 = pltpu.async_copy(b_hbm.at[lbl_v], tb_v, sem3)
        c4 = pltpu.async_copy(b_hbm.at[smp_v], sb_v, sem4)
        c1.wait()
        c2.wait()
        c3.wait()
        c4.wait()
        pltpu.sync_copy(tw_v, tw_out.at[pl.ds(bt, _TRUE_PER_W)])
        pltpu.sync_copy(sw_v, sw_out.at[pl.ds(bs, _SAMP_PER_W)])
        pltpu.sync_copy(tb_v, tb_out.at[pl.ds(bt, _TRUE_PER_W)])
        pltpu.sync_copy(sb_v, sb_out.at[pl.ds(bs, _SAMP_PER_W)])

    return gather_kernel(weights, biases, labels_i32, sampled_i32)


_BB = 256
_NBLK = BATCH // _BB


def _tc_loss_body(pred_ref, tw_ref, tb_ref, lbl_ref, sw_ref, sshift_ref,
                  out_ref):
    i = pl.program_id(0)
    pred = pred_ref[...]
    sw = sw_ref[...]
    logits = lax.dot_general(
        pred, sw, (((1,), (1,)), ((), ())),
        preferred_element_type=jnp.float32)
    logits = logits + sshift_ref[...]

    tw = tw_ref[...]
    tlogit = jnp.sum(pred * tw, axis=1, keepdims=True) + tb_ref[...]
    lblf = lbl_ref[...].astype(jnp.float32)
    p_true = jnp.log((lblf + 2.0) / (lblf + 1.0)) * (1.0 / _LOGV1)
    p = p_true
    log1p_neg = -p * (1.0 + p * (1.0 / 2.0 + p * (1.0 / 3.0 + p * (
        1.0 / 4.0 + p * (1.0 / 5.0 + p * (1.0 / 6.0 + p * (1.0 / 7.0)))))))
    x = NUM_SAMPLED * log1p_neg
    xs = jnp.maximum(x, -0.5)
    em1_series = xs * (1.0 + xs * (1.0 / 2.0 + xs * (1.0 / 6.0 + xs * (
        1.0 / 24.0 + xs * (1.0 / 120.0 + xs * (1.0 / 720.0 + xs * (
            1.0 / 5040.0)))))))
    em1 = jnp.where(x < -0.5, jnp.exp(x) - 1.0, em1_series)
    true_expected = -em1
    tlogit = tlogit - jnp.log(true_expected)

    m = jnp.maximum(jnp.max(logits, axis=1, keepdims=True), tlogit)
    se = jnp.sum(jnp.exp(logits - m), axis=1, keepdims=True) \
        + jnp.exp(tlogit - m)
    per_ex = m + jnp.log(se) - tlogit

    @pl.when(i == 0)
    def _init():
        out_ref[...] = jnp.zeros_like(out_ref)

    out_ref[...] += jnp.sum(per_ex) * (1.0 / BATCH)


def _tc_loss(predictions, true_w, true_b, labels_i32, samp_w, samp_shift):
    return pl.pallas_call(
        _tc_loss_body,
        grid=(_NBLK,),
        in_specs=[
            pl.BlockSpec((_BB, EMBED), lambda i: (i, 0)),
            pl.BlockSpec((_BB, EMBED), lambda i: (i, 0)),
            pl.BlockSpec((_BB, 1), lambda i: (i, 0)),
            pl.BlockSpec((_BB, 1), lambda i: (i, 0)),
            pl.BlockSpec((NUM_SAMPLED, EMBED), lambda i: (0, 0)),
            pl.BlockSpec((1, NUM_SAMPLED), lambda i: (0, 0)),
        ],
        out_specs=pl.BlockSpec((1, 1), lambda i: (0, 0)),
        out_shape=jax.ShapeDtypeStruct((1, 1), jnp.float32),
        compiler_params=pltpu.CompilerParams(
            dimension_semantics=("arbitrary",)),
    )(predictions, true_w, true_b, labels_i32, samp_w, samp_shift)


def kernel(predictions, labels, weights, biases):
    labels_flat = labels.reshape(-1).astype(jnp.int32)

    skey = jax.random.key(42)
    u = jax.random.uniform(skey, (NUM_SAMPLED,), dtype=jnp.float32)
    sampled = jnp.clip(
        (jnp.exp(u * jnp.log(VOCAB + 1.0)) - 1.0).astype(jnp.int32),
        0, VOCAB - 1)

    true_w, samp_w, true_b, samp_b = _sc_gather(
        weights, biases, labels_flat, sampled)

    sampf = sampled.astype(jnp.float32)
    p_samp = jnp.log((sampf + 2.0) / (sampf + 1.0)) / _LOGV1
    samp_expected = -jnp.expm1(NUM_SAMPLED * jnp.log1p(-p_samp))
    samp_shift = (samp_b - jnp.log(samp_expected)).reshape(1, NUM_SAMPLED)

    loss = _tc_loss(predictions, true_w, true_b.reshape(BATCH, 1),
                    labels_flat.reshape(BATCH, 1), samp_w, samp_shift)
    return loss[0, 0]
